# Initial kernel scaffold; baseline (speedup 1.0000x reference)
import jax
import jax.numpy as jnp
from jax.experimental import pallas as pl


def kernel(features, edge_index, W1, b1, W2, b2, W3, b3, noise):
    raise NotImplementedError("write your pallas kernel here")


# speedup vs baseline: 2.3401x; 2.3401x over previous
import functools

import jax
import jax.numpy as jnp
from jax import lax
from jax.experimental import pallas as pl
from jax.experimental.pallas import tpu as pltpu
from jax.experimental.pallas import tpu_sc as plsc

N = 10000
E = 160000
D_IN = 256
DH = 128
H2 = 128

NC = 2
NS = 16
BATCH = 128
EPT = 10112
NB = EPT // BATCH
EPAD = EPT * NS
PAD_ROW = 10008
NACC = 10240
RPT_A = NACC // NS
RPT_W = 624
TAIL = N - NS * RPT_W

RB = 1000
GRID_R = N // RB
BT = 1024
GRID_D = (N + BT - 1) // BT

_f32 = jnp.float32


def _deg_body(src_hbm, dst_hbm, zeros_hbm, ones_hbm, odeg_hbm, ideg_hbm,
              acc, idxv, onesv):
    c = lax.axis_index("c")
    s = lax.axis_index("s")
    pltpu.sync_copy(zeros_hbm.at[pl.ds(s * RPT_A, RPT_A)],
                    acc.at[pl.ds(s * RPT_A, RPT_A)])
    pltpu.sync_copy(ones_hbm, onesv)
    plsc.subcore_barrier()

    def run(idx_hbm):
        def body(b, carry):
            off = s * EPT + b * BATCH
            pltpu.sync_copy(idx_hbm.at[pl.ds(off, BATCH)], idxv)
            pltpu.sync_copy(onesv, acc.at[idxv], add=True)
            return carry
        lax.fori_loop(0, NB, body, 0)

    pl.when(c == 0)(lambda: run(src_hbm))
    pl.when(c == 1)(lambda: run(dst_hbm))
    plsc.subcore_barrier()

    def wout(o_hbm):
        pltpu.sync_copy(acc.at[pl.ds(s * RPT_W, RPT_W)],
                        o_hbm.at[pl.ds(s * RPT_W, RPT_W)])
        pl.when(s == NS - 1)(lambda: pltpu.sync_copy(
            acc.at[pl.ds(NS * RPT_W, TAIL)], o_hbm.at[pl.ds(NS * RPT_W, TAIL)]))

    pl.when(c == 0)(lambda: wout(odeg_hbm))
    pl.when(c == 1)(lambda: wout(ideg_hbm))


@functools.cache
def _deg_call():
    mesh = plsc.VectorSubcoreMesh(
        core_axis_name="c", subcore_axis_name="s",
        num_cores=NC, num_subcores=NS)
    return pl.kernel(
        _deg_body,
        out_type=(jax.ShapeDtypeStruct((N, 16), _f32),
                  jax.ShapeDtypeStruct((N, 16), _f32)),
        mesh=mesh,
        scratch_types=[
            pltpu.VMEM_SHARED((NACC, 16), _f32),
            pltpu.VMEM((BATCH,), jnp.int32),
            pltpu.VMEM((BATCH, 16), _f32),
        ],
    )


def _prop_body(src_hbm, dst_hbm, xa_hbm, xb_hbm, zeros_hbm, outa_hbm, outb_hbm,
               acc, sidx, didx, rows, gsem):
    c = lax.axis_index("c")
    s = lax.axis_index("s")
    pltpu.sync_copy(zeros_hbm.at[pl.ds(s * RPT_A, RPT_A)],
                    acc.at[pl.ds(s * RPT_A, RPT_A)])
    plsc.subcore_barrier()

    def run(x_hbm):
        def body(b, carry):
            off = s * EPT + b * BATCH
            pltpu.sync_copy(src_hbm.at[pl.ds(off, BATCH)], sidx)
            pltpu.sync_copy(dst_hbm.at[pl.ds(off, BATCH)], didx)
            pltpu.async_copy(x_hbm.at[sidx], rows, gsem).wait()
            pltpu.sync_copy(rows, acc.at[didx], add=True)
            return carry
        lax.fori_loop(0, NB, body, 0)

    pl.when(c == 0)(lambda: run(xa_hbm))
    pl.when(c == 1)(lambda: run(xb_hbm))
    plsc.subcore_barrier()

    def wout(o_hbm):
        pltpu.sync_copy(acc.at[pl.ds(s * RPT_W, RPT_W)],
                        o_hbm.at[pl.ds(s * RPT_W, RPT_W)])
        pl.when(s == NS - 1)(lambda: pltpu.sync_copy(
            acc.at[pl.ds(NS * RPT_W, TAIL)], o_hbm.at[pl.ds(NS * RPT_W, TAIL)]))

    pl.when(c == 0)(lambda: wout(outa_hbm))
    pl.when(c == 1)(lambda: wout(outb_hbm))


@functools.cache
def _prop_call():
    mesh = plsc.VectorSubcoreMesh(
        core_axis_name="c", subcore_axis_name="s",
        num_cores=NC, num_subcores=NS)
    return pl.kernel(
        _prop_body,
        out_type=(jax.ShapeDtypeStruct((N, DH), _f32),
                  jax.ShapeDtypeStruct((N, DH), _f32)),
        mesh=mesh,
        scratch_types=[
            pltpu.VMEM_SHARED((NACC, DH), _f32),
            pltpu.VMEM((BATCH,), jnp.int32),
            pltpu.VMEM((BATCH,), jnp.int32),
            pltpu.VMEM((BATCH, DH), _f32),
            pltpu.SemaphoreType.DMA,
        ],
    )


def _scale_body(x_ref, odeg_ref, xa_ref, xb_ref):
    a = lax.rsqrt(jnp.maximum(odeg_ref[:, 0:1], 1.0))
    xs = x_ref[...] * a
    xa_ref[...] = xs[:, :DH]
    xb_ref[...] = xs[:, DH:]


_scale_call = pl.pallas_call(
    _scale_body,
    grid=(GRID_R,),
    in_specs=[
        pl.BlockSpec((RB, D_IN), lambda i: (i, 0)),
        pl.BlockSpec((RB, 16), lambda i: (i, 0)),
    ],
    out_specs=[
        pl.BlockSpec((RB, DH), lambda i: (i, 0)),
        pl.BlockSpec((RB, DH), lambda i: (i, 0)),
    ],
    out_shape=(jax.ShapeDtypeStruct((N, DH), _f32),
               jax.ShapeDtypeStruct((N, DH), _f32)),
)


def _layer1_body(s1a_ref, s1b_ref, odeg_ref, ideg_ref, w_ref, b_ref,
                 ha_ref, hb_ref):
    cc = lax.rsqrt(jnp.maximum(ideg_ref[:, 0:1], 1.0))
    s1 = jnp.concatenate([s1a_ref[...], s1b_ref[...]], axis=1) * cc
    h = lax.dot_general(s1, w_ref[...], (((1,), (0,)), ((), ())),
                        precision=lax.Precision.HIGHEST,
                        preferred_element_type=_f32)
    h = jnp.maximum(h + b_ref[...], 0.0)
    a = lax.rsqrt(jnp.maximum(odeg_ref[:, 0:1], 1.0))
    hs = h * a
    ha_ref[...] = hs[:, :DH]
    hb_ref[...] = hs[:, DH:]


_layer1_call = pl.pallas_call(
    _layer1_body,
    grid=(GRID_R,),
    in_specs=[
        pl.BlockSpec((RB, DH), lambda i: (i, 0)),
        pl.BlockSpec((RB, DH), lambda i: (i, 0)),
        pl.BlockSpec((RB, 16), lambda i: (i, 0)),
        pl.BlockSpec((RB, 16), lambda i: (i, 0)),
        pl.BlockSpec((D_IN, D_IN), lambda i: (0, 0)),
        pl.BlockSpec((1, D_IN), lambda i: (0, 0)),
    ],
    out_specs=[
        pl.BlockSpec((RB, DH), lambda i: (i, 0)),
        pl.BlockSpec((RB, DH), lambda i: (i, 0)),
    ],
    out_shape=(jax.ShapeDtypeStruct((N, DH), _f32),
               jax.ShapeDtypeStruct((N, DH), _f32)),
)


def _z_body(s2a_ref, s2b_ref, ideg_ref, w_ref, b_ref, noise_ref, z_ref):
    cc = lax.rsqrt(jnp.maximum(ideg_ref[:, 0:1], 1.0))
    p = jnp.concatenate([s2a_ref[...], s2b_ref[...]], axis=1) * cc
    q = lax.dot_general(p, w_ref[...], (((1,), (0,)), ((), ())),
                        precision=lax.Precision.HIGHEST,
                        preferred_element_type=_f32)
    q = q + b_ref[...]
    z_ref[...] = q[:, :H2] + noise_ref[...] * jnp.exp(q[:, H2:])


_z_call = pl.pallas_call(
    _z_body,
    grid=(GRID_R,),
    in_specs=[
        pl.BlockSpec((RB, DH), lambda i: (i, 0)),
        pl.BlockSpec((RB, DH), lambda i: (i, 0)),
        pl.BlockSpec((RB, 16), lambda i: (i, 0)),
        pl.BlockSpec((D_IN, D_IN), lambda i: (0, 0)),
        pl.BlockSpec((1, D_IN), lambda i: (0, 0)),
        pl.BlockSpec((RB, H2), lambda i: (i, 0)),
    ],
    out_specs=pl.BlockSpec((RB, H2), lambda i: (i, 0)),
    out_shape=jax.ShapeDtypeStruct((N, H2), _f32),
)


def _dec_body(zl_ref, zr_ref, o_ref):
    acc = lax.dot_general(zl_ref[...], zr_ref[...], (((1,), (1,)), ((), ())),
                          precision=lax.Precision.HIGHEST,
                          preferred_element_type=_f32)
    o_ref[...] = jax.nn.sigmoid(acc)


_dec_call = pl.pallas_call(
    _dec_body,
    grid=(GRID_D, GRID_D),
    in_specs=[
        pl.BlockSpec((BT, H2), lambda i, j: (i, 0)),
        pl.BlockSpec((BT, H2), lambda i, j: (j, 0)),
    ],
    out_specs=pl.BlockSpec((BT, BT), lambda i, j: (i, j)),
    out_shape=jax.ShapeDtypeStruct((N, N), _f32),
)


def kernel(features, edge_index, W1, b1, W2, b2, W3, b3, noise):
    src = edge_index[0]
    dst = edge_index[1]
    trash = jnp.full((EPAD - E,), PAD_ROW, jnp.int32)
    src_prop = jnp.concatenate([src, jnp.zeros((EPAD - E,), jnp.int32)])
    src_deg = jnp.concatenate([src, trash])
    dst_pad = jnp.concatenate([dst, trash])

    zeros_acc = jnp.zeros((NACC, DH), _f32)
    zeros16 = jnp.zeros((NACC, 16), _f32)
    ones16 = jnp.ones((BATCH, 16), _f32)

    odeg, ideg = _deg_call()(src_deg, dst_pad, zeros16, ones16)
    xa, xb = _scale_call(features, odeg)
    s1a, s1b = _prop_call()(src_prop, dst_pad, xa, xb, zeros_acc)
    ha, hb = _layer1_call(s1a, s1b, odeg, ideg, W1, b1.reshape(1, -1))
    s2a, s2b = _prop_call()(src_prop, dst_pad, ha, hb, zeros_acc)

    W23 = jnp.concatenate([W2, W3], axis=1)
    b23 = jnp.concatenate([b2, b3]).reshape(1, -1)
    z = _z_call(s2a, s2b, ideg, W23, b23, noise)
    return _dec_call(z, z)


# speedup vs baseline: 2.4081x; 1.0291x over previous
import functools

import jax
import jax.numpy as jnp
from jax import lax
from jax.experimental import pallas as pl
from jax.experimental.pallas import tpu as pltpu
from jax.experimental.pallas import tpu_sc as plsc

N = 10000
E = 160000
D_IN = 256
DH = 128
H2 = 128

NC = 2
NS = 16
BATCH = 128
NB = 80
EPT = NB * BATCH
EPAD = EPT * NS
PAD_ROW = 10008
NACC = 10112
CH = 16
NCH = NB // CH
RPT_A = NACC // NS
RPT_W = 624
TAIL = N - NS * RPT_W

RB = 1000
GRID_R = N // RB
BT = 1024
GRID_D = (N + BT - 1) // BT

_f32 = jnp.float32


def _deg_body(src_hbm, dst_hbm, zeros_hbm, ones_hbm, odeg_hbm, ideg_hbm,
              acc, idx_all, onesv):
    c = lax.axis_index("c")
    s = lax.axis_index("s")
    pltpu.sync_copy(zeros_hbm.at[pl.ds(s * RPT_A, RPT_A)],
                    acc.at[pl.ds(s * RPT_A, RPT_A)])
    pltpu.sync_copy(ones_hbm, onesv)

    def run(idx_hbm):
        pltpu.sync_copy(idx_hbm.at[pl.ds(s * NB, NB)], idx_all)
        plsc.subcore_barrier()

        def body(b, carry):
            pltpu.sync_copy(onesv, acc.at[idx_all.at[b]], add=True)
            return carry
        lax.fori_loop(0, NB, body, 0)

    pl.when(c == 0)(lambda: run(src_hbm))
    pl.when(c == 1)(lambda: run(dst_hbm))
    plsc.subcore_barrier()

    def wout(o_hbm):
        pltpu.sync_copy(acc.at[pl.ds(s * RPT_W, RPT_W)],
                        o_hbm.at[pl.ds(s * RPT_W, RPT_W)])
        pl.when(s == NS - 1)(lambda: pltpu.sync_copy(
            acc.at[pl.ds(NS * RPT_W, TAIL)], o_hbm.at[pl.ds(NS * RPT_W, TAIL)]))

    pl.when(c == 0)(lambda: wout(odeg_hbm))
    pl.when(c == 1)(lambda: wout(ideg_hbm))


@functools.cache
def _deg_call():
    mesh = plsc.VectorSubcoreMesh(
        core_axis_name="c", subcore_axis_name="s",
        num_cores=NC, num_subcores=NS)
    return pl.kernel(
        _deg_body,
        out_type=(jax.ShapeDtypeStruct((N, 16), _f32),
                  jax.ShapeDtypeStruct((N, 16), _f32)),
        mesh=mesh,
        scratch_types=[
            pltpu.VMEM_SHARED((NACC, 16), _f32),
            pltpu.VMEM((NB, BATCH), jnp.int32),
            pltpu.VMEM((BATCH, 16), _f32),
        ],
    )


def _prop_body(src_hbm, dst_hbm, xa_hbm, xb_hbm, zeros_hbm, outa_hbm, outb_hbm,
               acc, sidx_all, didx_all, rows0, rows1, gsem):
    c = lax.axis_index("c")
    s = lax.axis_index("s")
    pltpu.sync_copy(zeros_hbm.at[pl.ds(s * RPT_A, RPT_A)],
                    acc.at[pl.ds(s * RPT_A, RPT_A)])

    plsc.subcore_barrier()

    def run(x_hbm):
        def chunk(g, carry):
            base = s * NB + g * CH
            pltpu.sync_copy(src_hbm.at[pl.ds(base, CH)], sidx_all)
            pltpu.sync_copy(dst_hbm.at[pl.ds(base, CH)], didx_all)
            pltpu.async_copy(x_hbm.at[sidx_all.at[0]], rows0, gsem)

            def body(i, carry2):
                b0 = 2 * i
                pltpu.make_async_copy(x_hbm.at[sidx_all.at[b0]], rows0,
                                      gsem).wait()
                pltpu.async_copy(x_hbm.at[sidx_all.at[b0 + 1]], rows1, gsem)
                pltpu.sync_copy(rows0, acc.at[didx_all.at[b0]], add=True)
                pltpu.make_async_copy(x_hbm.at[sidx_all.at[b0 + 1]], rows1,
                                      gsem).wait()

                def next_gather():
                    pltpu.async_copy(x_hbm.at[sidx_all.at[b0 + 2]], rows0,
                                     gsem)
                pl.when(i < CH // 2 - 1)(next_gather)
                pltpu.sync_copy(rows1, acc.at[didx_all.at[b0 + 1]], add=True)
                return carry2
            lax.fori_loop(0, CH // 2, body, 0)
            return carry
        lax.fori_loop(0, NCH, chunk, 0)

    pl.when(c == 0)(lambda: run(xa_hbm))
    pl.when(c == 1)(lambda: run(xb_hbm))
    plsc.subcore_barrier()

    def wout(o_hbm):
        pltpu.sync_copy(acc.at[pl.ds(s * RPT_W, RPT_W)],
                        o_hbm.at[pl.ds(s * RPT_W, RPT_W)])
        pl.when(s == NS - 1)(lambda: pltpu.sync_copy(
            acc.at[pl.ds(NS * RPT_W, TAIL)], o_hbm.at[pl.ds(NS * RPT_W, TAIL)]))

    pl.when(c == 0)(lambda: wout(outa_hbm))
    pl.when(c == 1)(lambda: wout(outb_hbm))


@functools.cache
def _prop_call():
    mesh = plsc.VectorSubcoreMesh(
        core_axis_name="c", subcore_axis_name="s",
        num_cores=NC, num_subcores=NS)
    return pl.kernel(
        _prop_body,
        out_type=(jax.ShapeDtypeStruct((N, DH), _f32),
                  jax.ShapeDtypeStruct((N, DH), _f32)),
        mesh=mesh,
        scratch_types=[
            pltpu.VMEM_SHARED((NACC, DH), _f32),
            pltpu.VMEM((CH, BATCH), jnp.int32),
            pltpu.VMEM((CH, BATCH), jnp.int32),
            pltpu.VMEM((BATCH, DH), _f32),
            pltpu.VMEM((BATCH, DH), _f32),
            pltpu.SemaphoreType.DMA,
        ],
    )


def _scale_body(x_ref, odeg_ref, xa_ref, xb_ref):
    a = lax.rsqrt(jnp.maximum(odeg_ref[:, 0:1], 1.0))
    xs = x_ref[...] * a
    xa_ref[...] = xs[:, :DH]
    xb_ref[...] = xs[:, DH:]


_scale_call = pl.pallas_call(
    _scale_body,
    grid=(GRID_R,),
    in_specs=[
        pl.BlockSpec((RB, D_IN), lambda i: (i, 0)),
        pl.BlockSpec((RB, 16), lambda i: (i, 0)),
    ],
    out_specs=[
        pl.BlockSpec((RB, DH), lambda i: (i, 0)),
        pl.BlockSpec((RB, DH), lambda i: (i, 0)),
    ],
    out_shape=(jax.ShapeDtypeStruct((N, DH), _f32),
               jax.ShapeDtypeStruct((N, DH), _f32)),
)


def _layer1_body(s1a_ref, s1b_ref, odeg_ref, ideg_ref, w_ref, b_ref,
                 ha_ref, hb_ref):
    cc = lax.rsqrt(jnp.maximum(ideg_ref[:, 0:1], 1.0))
    s1 = jnp.concatenate([s1a_ref[...], s1b_ref[...]], axis=1) * cc
    h = lax.dot_general(s1, w_ref[...], (((1,), (0,)), ((), ())),
                        precision=lax.Precision.HIGHEST,
                        preferred_element_type=_f32)
    h = jnp.maximum(h + b_ref[...], 0.0)
    a = lax.rsqrt(jnp.maximum(odeg_ref[:, 0:1], 1.0))
    hs = h * a
    ha_ref[...] = hs[:, :DH]
    hb_ref[...] = hs[:, DH:]


_layer1_call = pl.pallas_call(
    _layer1_body,
    grid=(GRID_R,),
    in_specs=[
        pl.BlockSpec((RB, DH), lambda i: (i, 0)),
        pl.BlockSpec((RB, DH), lambda i: (i, 0)),
        pl.BlockSpec((RB, 16), lambda i: (i, 0)),
        pl.BlockSpec((RB, 16), lambda i: (i, 0)),
        pl.BlockSpec((D_IN, D_IN), lambda i: (0, 0)),
        pl.BlockSpec((1, D_IN), lambda i: (0, 0)),
    ],
    out_specs=[
        pl.BlockSpec((RB, DH), lambda i: (i, 0)),
        pl.BlockSpec((RB, DH), lambda i: (i, 0)),
    ],
    out_shape=(jax.ShapeDtypeStruct((N, DH), _f32),
               jax.ShapeDtypeStruct((N, DH), _f32)),
)


def _z_body(s2a_ref, s2b_ref, ideg_ref, w_ref, b_ref, noise_ref, z_ref):
    cc = lax.rsqrt(jnp.maximum(ideg_ref[:, 0:1], 1.0))
    p = jnp.concatenate([s2a_ref[...], s2b_ref[...]], axis=1) * cc
    q = lax.dot_general(p, w_ref[...], (((1,), (0,)), ((), ())),
                        precision=lax.Precision.HIGHEST,
                        preferred_element_type=_f32)
    q = q + b_ref[...]
    z_ref[...] = q[:, :H2] + noise_ref[...] * jnp.exp(q[:, H2:])


_z_call = pl.pallas_call(
    _z_body,
    grid=(GRID_R,),
    in_specs=[
        pl.BlockSpec((RB, DH), lambda i: (i, 0)),
        pl.BlockSpec((RB, DH), lambda i: (i, 0)),
        pl.BlockSpec((RB, 16), lambda i: (i, 0)),
        pl.BlockSpec((D_IN, D_IN), lambda i: (0, 0)),
        pl.BlockSpec((1, D_IN), lambda i: (0, 0)),
        pl.BlockSpec((RB, H2), lambda i: (i, 0)),
    ],
    out_specs=pl.BlockSpec((RB, H2), lambda i: (i, 0)),
    out_shape=jax.ShapeDtypeStruct((N, H2), _f32),
)


def _dec_body(zl_ref, zr_ref, o_ref):
    acc = lax.dot_general(zl_ref[...], zr_ref[...], (((1,), (1,)), ((), ())),
                          precision=lax.Precision.HIGHEST,
                          preferred_element_type=_f32)
    o_ref[...] = jax.nn.sigmoid(acc)


_dec_call = pl.pallas_call(
    _dec_body,
    grid=(GRID_D, GRID_D),
    in_specs=[
        pl.BlockSpec((BT, H2), lambda i, j: (i, 0)),
        pl.BlockSpec((BT, H2), lambda i, j: (j, 0)),
    ],
    out_specs=pl.BlockSpec((BT, BT), lambda i, j: (i, j)),
    out_shape=jax.ShapeDtypeStruct((N, N), _f32),
)


def kernel(features, edge_index, W1, b1, W2, b2, W3, b3, noise):
    src = edge_index[0]
    dst = edge_index[1]
    trash = jnp.full((EPAD - E,), PAD_ROW, jnp.int32)
    src_prop = jnp.concatenate(
        [src, jnp.zeros((EPAD - E,), jnp.int32)]).reshape(NS * NB, BATCH)
    src_deg = jnp.concatenate([src, trash]).reshape(NS * NB, BATCH)
    dst_pad = jnp.concatenate([dst, trash]).reshape(NS * NB, BATCH)

    zeros_acc = jnp.zeros((NACC, DH), _f32)
    zeros16 = jnp.zeros((NACC, 16), _f32)
    ones16 = jnp.ones((BATCH, 16), _f32)

    odeg, ideg = _deg_call()(src_deg, dst_pad, zeros16, ones16)
    xa, xb = _scale_call(features, odeg)
    s1a, s1b = _prop_call()(src_prop, dst_pad, xa, xb, zeros_acc)
    ha, hb = _layer1_call(s1a, s1b, odeg, ideg, W1, b1.reshape(1, -1))
    s2a, s2b = _prop_call()(src_prop, dst_pad, ha, hb, zeros_acc)

    W23 = jnp.concatenate([W2, W3], axis=1)
    b23 = jnp.concatenate([b2, b3]).reshape(1, -1)
    z = _z_call(s2a, s2b, ideg, W23, b23, noise)
    return _dec_call(z, z)


# speedup vs baseline: 2.7577x; 1.1452x over previous
import functools

import jax
import jax.numpy as jnp
from jax import lax
from jax.experimental import pallas as pl
from jax.experimental.pallas import tpu as pltpu
from jax.experimental.pallas import tpu_sc as plsc

N = 10000
E = 160000
D_IN = 256
DH = 128
H2 = 128

NC = 2
NS = 16
BATCH = 128
NB = 80
EPT = NB * BATCH
EPAD = EPT * NS
PAD_ROW = 10008
NACC = 10112
CH = 16
NCH = NB // CH
RPT_A = NACC // NS
RPT_W = 624
TAIL = N - NS * RPT_W

RB = 1000
GRID_R = N // RB
BT = 1024
GRID_D = (N + BT - 1) // BT

_f32 = jnp.float32


def _deg_body(src_hbm, dst_hbm, zeros_hbm, ones_hbm, odeg_hbm, ideg_hbm,
              acc, idx_all, onesv):
    c = lax.axis_index("c")
    s = lax.axis_index("s")
    pltpu.sync_copy(zeros_hbm.at[pl.ds(s * RPT_A, RPT_A)],
                    acc.at[pl.ds(s * RPT_A, RPT_A)])
    pltpu.sync_copy(ones_hbm, onesv)

    def run(idx_hbm):
        pltpu.sync_copy(idx_hbm.at[pl.ds(s * NB, NB)], idx_all)
        plsc.subcore_barrier()

        def body(b, carry):
            pltpu.sync_copy(onesv, acc.at[idx_all.at[b]], add=True)
            return carry
        lax.fori_loop(0, NB, body, 0)

    pl.when(c == 0)(lambda: run(src_hbm))
    pl.when(c == 1)(lambda: run(dst_hbm))
    plsc.subcore_barrier()

    def wout(o_hbm):
        pltpu.sync_copy(acc.at[pl.ds(s * RPT_W, RPT_W)],
                        o_hbm.at[pl.ds(s * RPT_W, RPT_W)])
        pl.when(s == NS - 1)(lambda: pltpu.sync_copy(
            acc.at[pl.ds(NS * RPT_W, TAIL)], o_hbm.at[pl.ds(NS * RPT_W, TAIL)]))

    pl.when(c == 0)(lambda: wout(odeg_hbm))
    pl.when(c == 1)(lambda: wout(ideg_hbm))


@functools.cache
def _deg_call():
    mesh = plsc.VectorSubcoreMesh(
        core_axis_name="c", subcore_axis_name="s",
        num_cores=NC, num_subcores=NS)
    return pl.kernel(
        _deg_body,
        out_type=(jax.ShapeDtypeStruct((N, 16), _f32),
                  jax.ShapeDtypeStruct((N, 16), _f32)),
        mesh=mesh,
        scratch_types=[
            pltpu.VMEM_SHARED((NACC, 16), _f32),
            pltpu.VMEM((NB, BATCH), jnp.int32),
            pltpu.VMEM((BATCH, 16), _f32),
        ],
    )


def _prop_body(src_hbm, dst_hbm, xa_hbm, xb_hbm, zeros_hbm, outa_hbm, outb_hbm,
               acc, sidx_all, didx_all, rows0, rows1, gsem):
    c = lax.axis_index("c")
    s = lax.axis_index("s")
    pltpu.sync_copy(zeros_hbm.at[pl.ds(s * RPT_A, RPT_A)],
                    acc.at[pl.ds(s * RPT_A, RPT_A)])

    plsc.subcore_barrier()

    def run(x_hbm):
        def chunk(g, carry):
            base = s * NB + g * CH
            pltpu.sync_copy(src_hbm.at[pl.ds(base, CH)], sidx_all)
            pltpu.sync_copy(dst_hbm.at[pl.ds(base, CH)], didx_all)
            pltpu.async_copy(x_hbm.at[sidx_all.at[0]], rows0, gsem)

            def body(i, carry2):
                b0 = 2 * i
                pltpu.make_async_copy(x_hbm.at[sidx_all.at[b0]], rows0,
                                      gsem).wait()
                pltpu.async_copy(x_hbm.at[sidx_all.at[b0 + 1]], rows1, gsem)
                pltpu.sync_copy(rows0, acc.at[didx_all.at[b0]], add=True)
                pltpu.make_async_copy(x_hbm.at[sidx_all.at[b0 + 1]], rows1,
                                      gsem).wait()

                def next_gather():
                    pltpu.async_copy(x_hbm.at[sidx_all.at[b0 + 2]], rows0,
                                     gsem)
                pl.when(i < CH // 2 - 1)(next_gather)
                pltpu.sync_copy(rows1, acc.at[didx_all.at[b0 + 1]], add=True)
                return carry2
            lax.fori_loop(0, CH // 2, body, 0)
            return carry
        lax.fori_loop(0, NCH, chunk, 0)

    pl.when(c == 0)(lambda: run(xa_hbm))
    pl.when(c == 1)(lambda: run(xb_hbm))
    plsc.subcore_barrier()

    def wout(o_hbm):
        pltpu.sync_copy(acc.at[pl.ds(s * RPT_W, RPT_W)],
                        o_hbm.at[pl.ds(s * RPT_W, RPT_W)])
        pl.when(s == NS - 1)(lambda: pltpu.sync_copy(
            acc.at[pl.ds(NS * RPT_W, TAIL)], o_hbm.at[pl.ds(NS * RPT_W, TAIL)]))

    pl.when(c == 0)(lambda: wout(outa_hbm))
    pl.when(c == 1)(lambda: wout(outb_hbm))


@functools.cache
def _prop_call():
    mesh = plsc.VectorSubcoreMesh(
        core_axis_name="c", subcore_axis_name="s",
        num_cores=NC, num_subcores=NS)
    return pl.kernel(
        _prop_body,
        out_type=(jax.ShapeDtypeStruct((N, DH), _f32),
                  jax.ShapeDtypeStruct((N, DH), _f32)),
        mesh=mesh,
        scratch_types=[
            pltpu.VMEM_SHARED((NACC, DH), _f32),
            pltpu.VMEM((CH, BATCH), jnp.int32),
            pltpu.VMEM((CH, BATCH), jnp.int32),
            pltpu.VMEM((BATCH, DH), _f32),
            pltpu.VMEM((BATCH, DH), _f32),
            pltpu.SemaphoreType.DMA,
        ],
    )


def _scale_body(x_ref, odeg_ref, xa_ref, xb_ref):
    a = lax.rsqrt(jnp.maximum(odeg_ref[:, 0:1], 1.0))
    xs = x_ref[...] * a
    xa_ref[...] = xs[:, :DH]
    xb_ref[...] = xs[:, DH:]


_scale_call = pl.pallas_call(
    _scale_body,
    grid=(GRID_R,),
    in_specs=[
        pl.BlockSpec((RB, D_IN), lambda i: (i, 0)),
        pl.BlockSpec((RB, 16), lambda i: (i, 0)),
    ],
    out_specs=[
        pl.BlockSpec((RB, DH), lambda i: (i, 0)),
        pl.BlockSpec((RB, DH), lambda i: (i, 0)),
    ],
    out_shape=(jax.ShapeDtypeStruct((N, DH), _f32),
               jax.ShapeDtypeStruct((N, DH), _f32)),
)


def _layer1_body(s1a_ref, s1b_ref, odeg_ref, ideg_ref, w_ref, b_ref,
                 ha_ref, hb_ref):
    cc = lax.rsqrt(jnp.maximum(ideg_ref[:, 0:1], 1.0))
    s1 = jnp.concatenate([s1a_ref[...], s1b_ref[...]], axis=1) * cc
    h = lax.dot_general(s1, w_ref[...], (((1,), (0,)), ((), ())),
                        precision=lax.Precision.HIGHEST,
                        preferred_element_type=_f32)
    h = jnp.maximum(h + b_ref[...], 0.0)
    a = lax.rsqrt(jnp.maximum(odeg_ref[:, 0:1], 1.0))
    hs = h * a
    ha_ref[...] = hs[:, :DH]
    hb_ref[...] = hs[:, DH:]


_layer1_call = pl.pallas_call(
    _layer1_body,
    grid=(GRID_R,),
    in_specs=[
        pl.BlockSpec((RB, DH), lambda i: (i, 0)),
        pl.BlockSpec((RB, DH), lambda i: (i, 0)),
        pl.BlockSpec((RB, 16), lambda i: (i, 0)),
        pl.BlockSpec((RB, 16), lambda i: (i, 0)),
        pl.BlockSpec((D_IN, D_IN), lambda i: (0, 0)),
        pl.BlockSpec((1, D_IN), lambda i: (0, 0)),
    ],
    out_specs=[
        pl.BlockSpec((RB, DH), lambda i: (i, 0)),
        pl.BlockSpec((RB, DH), lambda i: (i, 0)),
    ],
    out_shape=(jax.ShapeDtypeStruct((N, DH), _f32),
               jax.ShapeDtypeStruct((N, DH), _f32)),
)


def _z_body(s2a_ref, s2b_ref, ideg_ref, w_ref, b_ref, noise_ref, z_ref):
    cc = lax.rsqrt(jnp.maximum(ideg_ref[:, 0:1], 1.0))
    p = jnp.concatenate([s2a_ref[...], s2b_ref[...]], axis=1) * cc
    q = lax.dot_general(p, w_ref[...], (((1,), (0,)), ((), ())),
                        precision=lax.Precision.HIGHEST,
                        preferred_element_type=_f32)
    q = q + b_ref[...]
    z_ref[...] = q[:, :H2] + noise_ref[...] * jnp.exp(q[:, H2:])


_z_call = pl.pallas_call(
    _z_body,
    grid=(GRID_R,),
    in_specs=[
        pl.BlockSpec((RB, DH), lambda i: (i, 0)),
        pl.BlockSpec((RB, DH), lambda i: (i, 0)),
        pl.BlockSpec((RB, 16), lambda i: (i, 0)),
        pl.BlockSpec((D_IN, D_IN), lambda i: (0, 0)),
        pl.BlockSpec((1, D_IN), lambda i: (0, 0)),
        pl.BlockSpec((RB, H2), lambda i: (i, 0)),
    ],
    out_specs=pl.BlockSpec((RB, H2), lambda i: (i, 0)),
    out_shape=jax.ShapeDtypeStruct((N, H2), _f32),
)


def _dec_body(zl_ref, zr_ref, o_ref):
    zl = zl_ref[...]
    zr = zr_ref[...]
    lh = zl.astype(jnp.bfloat16)
    ll = (zl - lh.astype(_f32)).astype(jnp.bfloat16)
    rh = zr.astype(jnp.bfloat16)
    rl = (zr - rh.astype(_f32)).astype(jnp.bfloat16)
    dims = (((1,), (1,)), ((), ()))

    def bdot(a, b):
        return lax.dot_general(a, b, dims, preferred_element_type=_f32)

    acc = bdot(lh, rl) + bdot(ll, rh)
    acc = acc + bdot(lh, rh)
    o_ref[...] = jax.nn.sigmoid(acc)


_dec_call = pl.pallas_call(
    _dec_body,
    grid=(GRID_D, GRID_D),
    in_specs=[
        pl.BlockSpec((BT, H2), lambda i, j: (i, 0)),
        pl.BlockSpec((BT, H2), lambda i, j: (j, 0)),
    ],
    out_specs=pl.BlockSpec((BT, BT), lambda i, j: (i, j)),
    out_shape=jax.ShapeDtypeStruct((N, N), _f32),
)


def kernel(features, edge_index, W1, b1, W2, b2, W3, b3, noise):
    src = edge_index[0]
    dst = edge_index[1]
    trash = jnp.full((EPAD - E,), PAD_ROW, jnp.int32)
    src_prop = jnp.concatenate(
        [src, jnp.zeros((EPAD - E,), jnp.int32)]).reshape(NS * NB, BATCH)
    src_deg = jnp.concatenate([src, trash]).reshape(NS * NB, BATCH)
    dst_pad = jnp.concatenate([dst, trash]).reshape(NS * NB, BATCH)

    zeros_acc = jnp.zeros((NACC, DH), _f32)
    zeros16 = jnp.zeros((NACC, 16), _f32)
    ones16 = jnp.ones((BATCH, 16), _f32)

    odeg, ideg = _deg_call()(src_deg, dst_pad, zeros16, ones16)
    xa, xb = _scale_call(features, odeg)
    s1a, s1b = _prop_call()(src_prop, dst_pad, xa, xb, zeros_acc)
    ha, hb = _layer1_call(s1a, s1b, odeg, ideg, W1, b1.reshape(1, -1))
    s2a, s2b = _prop_call()(src_prop, dst_pad, ha, hb, zeros_acc)

    W23 = jnp.concatenate([W2, W3], axis=1)
    b23 = jnp.concatenate([b2, b3]).reshape(1, -1)
    z = _z_call(s2a, s2b, ideg, W23, b23, noise)
    return _dec_call(z, z)
